# Initial kernel scaffold; baseline (speedup 1.0000x reference)
import jax
import jax.numpy as jnp
from jax.experimental import pallas as pl


def kernel(board_tokens, color_tokens, piece_table, color_table, square_table, gamma, beta):
    raise NotImplementedError("write your pallas kernel here")


# speedup vs baseline: 4.5087x; 4.5087x over previous
import functools

import jax
import jax.numpy as jnp
from jax import lax
from jax.experimental import pallas as pl
from jax.experimental.pallas import tpu as pltpu
from jax.experimental.pallas import tpu_sc as plsc

D_MODEL = 256
SEQ = 65
EPS = 1e-5
N_PIECE = 8
N_COLOR = 3
N_ROWS = N_PIECE * N_COLOR * SEQ

NUM_CORES = 2
NUM_SUBCORES = 16
NUM_WORKERS = NUM_CORES * NUM_SUBCORES


def _table_and_idx_body(piece, color, square, gamma, beta, board, ctok,
                        table_o, idx_o):
    pv = piece[...]
    cv = color[...]
    sv = square[...]
    r = lax.broadcasted_iota(jnp.int32, (N_PIECE * N_COLOR, 1), 0)
    p = r // N_COLOR
    c = r - p * N_COLOR
    pc = jnp.zeros((N_PIECE * N_COLOR, D_MODEL), jnp.float32)
    for k in range(N_PIECE):
        pc = pc + jnp.where(p == k, 1.0, 0.0) * pv[k:k + 1, :]
    for k in range(N_COLOR):
        pc = pc + jnp.where(c == k, 1.0, 0.0) * cv[k:k + 1, :]
    x = pc[:, None, :] + sv[None, :, :]
    m = jnp.mean(x, axis=-1, keepdims=True)
    d = x - m
    v = jnp.mean(d * d, axis=-1, keepdims=True)
    y = d * lax.rsqrt(v + EPS)
    table_o[...] = y * gamma[...] + beta[...]
    bsz, seq = idx_o.shape
    s_iota = lax.broadcasted_iota(jnp.int32, (bsz, seq), 1)
    idx_o[...] = board[...] * (N_COLOR * SEQ) + ctok[...] * SEQ + s_iota


def _build_table_and_idx(piece, color, square, gamma, beta, board, ctok):
    bsz = board.shape[0]
    return pl.pallas_call(
        _table_and_idx_body,
        out_shape=(
            jax.ShapeDtypeStruct((N_PIECE * N_COLOR, SEQ, D_MODEL), jnp.float32),
            jax.ShapeDtypeStruct((bsz, SEQ), jnp.int32),
        ),
    )(piece, color, square, gamma.reshape(1, D_MODEL), beta.reshape(1, D_MODEL),
      board, ctok)


def _make_gather(total_rows):
    assert total_rows % (8 * NUM_WORKERS) == 0
    rows_per_worker = total_rows // NUM_WORKERS
    chunk = 128
    assert rows_per_worker % chunk == 0
    n_chunks = rows_per_worker // chunk
    mesh = plsc.VectorSubcoreMesh(core_axis_name="c", subcore_axis_name="s")

    @functools.partial(
        pl.kernel,
        mesh=mesh,
        out_type=jax.ShapeDtypeStruct((total_rows, D_MODEL), jnp.float32),
        scratch_types=[
            pltpu.VMEM((rows_per_worker,), jnp.int32),
            pltpu.VMEM((chunk, D_MODEL), jnp.float32),
            pltpu.SemaphoreType.DMA,
        ],
    )
    def gather(table_hbm, idx_hbm, out_hbm, idx_v, rows_v, sem):
        wid = lax.axis_index("s") * NUM_CORES + lax.axis_index("c")
        base = wid * rows_per_worker
        pltpu.sync_copy(idx_hbm.at[pl.ds(base, rows_per_worker)], idx_v)

        def body(i, carry):
            pltpu.async_copy(
                table_hbm.at[idx_v.at[pl.ds(i * chunk, chunk)]], rows_v, sem
            ).wait()
            pltpu.sync_copy(rows_v, out_hbm.at[pl.ds(base + i * chunk, chunk)])
            return carry

        lax.fori_loop(0, n_chunks, body, 0)

    return gather


def kernel(board_tokens, color_tokens, piece_table, color_table, square_table,
           gamma, beta):
    bsz = board_tokens.shape[0]
    total = bsz * SEQ
    table3, idx = _build_table_and_idx(
        piece_table, color_table, square_table, gamma, beta,
        board_tokens.astype(jnp.int32), color_tokens.astype(jnp.int32))
    table2 = table3.reshape(N_ROWS, D_MODEL)
    out = _make_gather(total)(table2, idx.reshape(total))
    return out.reshape(bsz, SEQ, D_MODEL)


# speedup vs baseline: 4.6168x; 1.0240x over previous
import functools

import jax
import jax.numpy as jnp
from jax import lax
from jax.experimental import pallas as pl
from jax.experimental.pallas import tpu as pltpu
from jax.experimental.pallas import tpu_sc as plsc

D_MODEL = 256
SEQ = 65
EPS = 1e-5
N_PIECE = 8
N_COLOR = 3
N_ROWS = N_PIECE * N_COLOR * SEQ

NUM_CORES = 2
NUM_SUBCORES = 16
NUM_WORKERS = NUM_CORES * NUM_SUBCORES


def _table_and_idx_body(piece, color, square, gamma, beta, board, ctok,
                        table_o, idx_o):
    pv = piece[...]
    cv = color[...]
    sv = square[...]
    r = lax.broadcasted_iota(jnp.int32, (N_PIECE * N_COLOR, 1), 0)
    p = r // N_COLOR
    c = r - p * N_COLOR
    pc = jnp.zeros((N_PIECE * N_COLOR, D_MODEL), jnp.float32)
    for k in range(N_PIECE):
        pc = pc + jnp.where(p == k, 1.0, 0.0) * pv[k:k + 1, :]
    for k in range(N_COLOR):
        pc = pc + jnp.where(c == k, 1.0, 0.0) * cv[k:k + 1, :]
    x = pc[:, None, :] + sv[None, :, :]
    m = jnp.mean(x, axis=-1, keepdims=True)
    d = x - m
    v = jnp.mean(d * d, axis=-1, keepdims=True)
    y = d * lax.rsqrt(v + EPS)
    table_o[...] = y * gamma[...] + beta[...]
    bsz, seq = idx_o.shape
    s_iota = lax.broadcasted_iota(jnp.int32, (bsz, seq), 1)
    idx_o[...] = board[...] * (N_COLOR * SEQ) + ctok[...] * SEQ + s_iota


def _build_table_and_idx(piece, color, square, gamma, beta, board, ctok):
    bsz = board.shape[0]
    return pl.pallas_call(
        _table_and_idx_body,
        out_shape=(
            jax.ShapeDtypeStruct((N_PIECE * N_COLOR, SEQ, D_MODEL), jnp.float32),
            jax.ShapeDtypeStruct((bsz, SEQ), jnp.int32),
        ),
    )(piece, color, square, gamma.reshape(1, D_MODEL), beta.reshape(1, D_MODEL),
      board, ctok)


def _make_gather(total_rows):
    assert total_rows % (8 * NUM_WORKERS) == 0
    rows_per_worker = total_rows // NUM_WORKERS
    chunk = 104
    assert rows_per_worker % (2 * chunk) == 0
    n_outer = rows_per_worker // (2 * chunk)
    mesh = plsc.VectorSubcoreMesh(core_axis_name="c", subcore_axis_name="s")

    @functools.partial(
        pl.kernel,
        mesh=mesh,
        out_type=jax.ShapeDtypeStruct((total_rows, D_MODEL), jnp.float32),
        scratch_types=[
            pltpu.VMEM((rows_per_worker,), jnp.int32),
            pltpu.VMEM((2, chunk, D_MODEL), jnp.float32),
            pltpu.SemaphoreType.DMA,
            pltpu.SemaphoreType.DMA,
            pltpu.SemaphoreType.DMA,
        ],
    )
    def gather(table_hbm, idx_hbm, out_hbm, idx_v, rows_v, gsem, wsem0, wsem1):
        wid = lax.axis_index("s") * NUM_CORES + lax.axis_index("c")
        base = wid * rows_per_worker
        pltpu.sync_copy(idx_hbm.at[pl.ds(base, rows_per_worker)], idx_v)

        def g_desc(i, b):
            return pltpu.make_async_copy(
                table_hbm.at[idx_v.at[pl.ds(i * chunk, chunk)]],
                rows_v.at[b], gsem)

        def w_desc(i, b, sem):
            return pltpu.make_async_copy(
                rows_v.at[b], out_hbm.at[pl.ds(base + i * chunk, chunk)], sem)

        g_desc(0, 0).start()

        def body(j, carry):
            i0 = 2 * j
            g_desc(i0, 0).wait()
            w_desc(i0, 0, wsem0).start()
            @pl.when(j > 0)
            def _():
                w_desc(i0 - 1, 1, wsem1).wait()
            g_desc(i0 + 1, 1).start()
            g_desc(i0 + 1, 1).wait()
            w_desc(i0 + 1, 1, wsem1).start()
            w_desc(i0, 0, wsem0).wait()
            @pl.when(j + 1 < n_outer)
            def _():
                g_desc(i0 + 2, 0).start()
            return carry

        lax.fori_loop(0, n_outer, body, 0)
        w_desc(2 * n_outer - 1, 1, wsem1).wait()

    return gather


def kernel(board_tokens, color_tokens, piece_table, color_table, square_table,
           gamma, beta):
    bsz = board_tokens.shape[0]
    total = bsz * SEQ
    table3, idx = _build_table_and_idx(
        piece_table, color_table, square_table, gamma, beta,
        board_tokens.astype(jnp.int32), color_tokens.astype(jnp.int32))
    table2 = table3.reshape(N_ROWS, D_MODEL)
    out = _make_gather(total)(table2, idx.reshape(total))
    return out.reshape(bsz, SEQ, D_MODEL)


# speedup vs baseline: 7.6294x; 1.6525x over previous
import functools

import jax
import jax.numpy as jnp
from jax import lax
from jax.experimental import pallas as pl
from jax.experimental.pallas import tpu as pltpu
from jax.experimental.pallas import tpu_sc as plsc

D_MODEL = 256
SEQ = 65
SEQ_PAD = 80
EPS = 1e-5
N_PIECE = 8
N_COLOR = 3
N_ROWS = N_PIECE * N_COLOR * SEQ

NUM_CORES = 2
NUM_SUBCORES = 16
NUM_WORKERS = NUM_CORES * NUM_SUBCORES


def _table_and_idx_body(piece, color, square, gamma, beta, board, ctok,
                        table_o, idx_o):
    pv = piece[...]
    cv = color[...]
    sv = square[...]
    r = lax.broadcasted_iota(jnp.int32, (N_PIECE * N_COLOR, 1), 0)
    p = r // N_COLOR
    c = r - p * N_COLOR
    pc = jnp.zeros((N_PIECE * N_COLOR, D_MODEL), jnp.float32)
    for k in range(N_PIECE):
        pc = pc + jnp.where(p == k, 1.0, 0.0) * pv[k:k + 1, :]
    for k in range(N_COLOR):
        pc = pc + jnp.where(c == k, 1.0, 0.0) * cv[k:k + 1, :]
    x = pc[:, None, :] + sv[None, :, :]
    m = jnp.mean(x, axis=-1, keepdims=True)
    d = x - m
    v = jnp.mean(d * d, axis=-1, keepdims=True)
    y = d * lax.rsqrt(v + EPS)
    table_o[...] = y * gamma[...] + beta[...]
    bsz = board.shape[0]
    s_iota = lax.broadcasted_iota(jnp.int32, (bsz, SEQ), 1)
    idx = board[...] * (N_COLOR * SEQ) + ctok[...] * SEQ + s_iota
    pad = jnp.zeros((bsz, SEQ_PAD - SEQ), jnp.int32)
    idx_o[...] = jnp.concatenate([idx, pad], axis=1)


def _build_table_and_idx(piece, color, square, gamma, beta, board, ctok):
    bsz = board.shape[0]
    return pl.pallas_call(
        _table_and_idx_body,
        out_shape=(
            jax.ShapeDtypeStruct((N_PIECE * N_COLOR, SEQ, D_MODEL), jnp.float32),
            jax.ShapeDtypeStruct((bsz, SEQ_PAD), jnp.int32),
        ),
    )(piece, color, square, gamma.reshape(1, D_MODEL), beta.reshape(1, D_MODEL),
      board, ctok)


def _make_gather(bsz):
    assert bsz % NUM_WORKERS == 0
    nb = bsz // NUM_WORKERS
    half = nb // 2
    assert half % 2 == 0
    mesh = plsc.VectorSubcoreMesh(core_axis_name="c", subcore_axis_name="s")

    @functools.partial(
        pl.kernel,
        mesh=mesh,
        out_type=jax.ShapeDtypeStruct((bsz, SEQ, D_MODEL), jnp.float32),
        scratch_types=[
            pltpu.VMEM((nb * SEQ_PAD,), jnp.int32),
            pltpu.VMEM((2, 64, D_MODEL), jnp.float32),
            pltpu.VMEM((2, 1, D_MODEL), jnp.float32),
            pltpu.SemaphoreType.DMA,
            pltpu.SemaphoreType.DMA,
            pltpu.SemaphoreType.DMA,
            pltpu.SemaphoreType.DMA,
        ],
    )
    def gather(table_hbm, idx_hbm, out_hbm, idx_v, rows_v, tail_v,
               gsem0, gsem1, wsem0, wsem1):
        wid = lax.axis_index("s") * NUM_CORES + lax.axis_index("c")
        base = wid * nb
        gsem = (gsem0, gsem1)
        wsem = (wsem0, wsem1)
        pltpu.sync_copy(idx_hbm.at[pl.ds(base * SEQ_PAD, nb * SEQ_PAD)], idx_v)

        def g_main(j, b):
            return pltpu.make_async_copy(
                table_hbm.at[idx_v.at[pl.ds(j * SEQ_PAD, 64)]],
                rows_v.at[b], gsem[b])

        def g_tail(j, b):
            return pltpu.make_async_copy(
                table_hbm.at[idx_v.at[pl.ds(j * SEQ_PAD + 64, 1)]],
                tail_v.at[b], gsem[b])

        def g_start(j, b):
            g_main(j, b).start()
            g_tail(j, b).start()

        def g_wait(j, b):
            g_main(j, b).wait()
            g_tail(j, b).wait()

        def w_main(j, b):
            return pltpu.make_async_copy(
                rows_v.at[b], out_hbm.at[base + j, pl.ds(0, 64)], wsem[b])

        def w_tail(j, b):
            return pltpu.make_async_copy(
                tail_v.at[b], out_hbm.at[base + j, pl.ds(64, 1)], wsem[b])

        def w_start(j, b):
            w_main(j, b).start()
            w_tail(j, b).start()

        def w_wait(j, b):
            w_main(j, b).wait()
            w_tail(j, b).wait()

        g_start(0, 0)
        g_start(1, 1)

        def body(i, carry):
            j0 = 2 * i
            more = j0 + 2 < nb
            g_wait(j0, 0)
            w_start(j0, 0)
            g_wait(j0 + 1, 1)
            w_start(j0 + 1, 1)
            w_wait(j0, 0)
            @pl.when(more)
            def _():
                g_start(j0 + 2, 0)
            w_wait(j0 + 1, 1)
            @pl.when(more)
            def _():
                g_start(j0 + 3, 1)
            return carry

        lax.fori_loop(0, nb // 2, body, 0)

    return gather


def kernel(board_tokens, color_tokens, piece_table, color_table, square_table,
           gamma, beta):
    bsz = board_tokens.shape[0]
    table3, idx = _build_table_and_idx(
        piece_table, color_table, square_table, gamma, beta,
        board_tokens.astype(jnp.int32), color_tokens.astype(jnp.int32))
    table2 = table3.reshape(N_ROWS, D_MODEL)
    return _make_gather(bsz)(table2, idx.reshape(bsz * SEQ_PAD))


# speedup vs baseline: 7.6492x; 1.0026x over previous
import functools

import jax
import jax.numpy as jnp
from jax import lax
from jax.experimental import pallas as pl
from jax.experimental.pallas import tpu as pltpu
from jax.experimental.pallas import tpu_sc as plsc

D_MODEL = 256
SEQ = 65
SEQ_PAD = 80
EPS = 1e-5
N_PIECE = 8
N_COLOR = 3
N_ROWS = N_PIECE * N_COLOR * SEQ

NUM_CORES = 2
NUM_SUBCORES = 16
NUM_WORKERS = NUM_CORES * NUM_SUBCORES


def _table_and_idx_body(piece, color, square, gamma, beta, board, ctok,
                        table_o, idx_o):
    pv = piece[...]
    cv = color[...]
    sv = square[...]
    r = lax.broadcasted_iota(jnp.int32, (N_PIECE * N_COLOR, 1), 0)
    p = r // N_COLOR
    c = r - p * N_COLOR
    pc = jnp.zeros((N_PIECE * N_COLOR, D_MODEL), jnp.float32)
    for k in range(N_PIECE):
        pc = pc + jnp.where(p == k, 1.0, 0.0) * pv[k:k + 1, :]
    for k in range(N_COLOR):
        pc = pc + jnp.where(c == k, 1.0, 0.0) * cv[k:k + 1, :]
    x = pc[:, None, :] + sv[None, :, :]
    m = jnp.mean(x, axis=-1, keepdims=True)
    d = x - m
    v = jnp.mean(d * d, axis=-1, keepdims=True)
    y = d * lax.rsqrt(v + EPS)
    table_o[...] = y * gamma[...] + beta[...]
    bsz = board.shape[0]
    s_iota = lax.broadcasted_iota(jnp.int32, (bsz, SEQ), 1)
    idx = board[...] * (N_COLOR * SEQ) + ctok[...] * SEQ + s_iota
    pad = jnp.zeros((bsz, SEQ_PAD - SEQ), jnp.int32)
    idx_o[...] = jnp.concatenate([idx, pad], axis=1)


def _build_table_and_idx(piece, color, square, gamma, beta, board, ctok):
    bsz = board.shape[0]
    return pl.pallas_call(
        _table_and_idx_body,
        out_shape=(
            jax.ShapeDtypeStruct((N_PIECE * N_COLOR, SEQ, D_MODEL), jnp.float32),
            jax.ShapeDtypeStruct((bsz, SEQ_PAD), jnp.int32),
        ),
    )(piece, color, square, gamma.reshape(1, D_MODEL), beta.reshape(1, D_MODEL),
      board, ctok)


def _make_gather(bsz):
    assert bsz % NUM_WORKERS == 0
    nb = bsz // NUM_WORKERS
    half = nb // 2
    assert half % 2 == 0
    mesh = plsc.VectorSubcoreMesh(core_axis_name="c", subcore_axis_name="s")

    @functools.partial(
        pl.kernel,
        mesh=mesh,
        out_type=jax.ShapeDtypeStruct((bsz, SEQ, D_MODEL), jnp.float32),
        scratch_types=[
            pltpu.VMEM((nb * SEQ_PAD,), jnp.int32),
            pltpu.VMEM((2, 64, D_MODEL), jnp.float32),
            pltpu.VMEM((2, 1, D_MODEL), jnp.float32),
            pltpu.SemaphoreType.DMA,
            pltpu.SemaphoreType.DMA,
            pltpu.SemaphoreType.DMA,
            pltpu.SemaphoreType.DMA,
        ],
        compiler_params=pltpu.CompilerParams(use_tc_tiling_on_sc=True),
    )
    def gather(table_hbm, idx_hbm, out_hbm, idx_v, rows_v, tail_v,
               gsem0, gsem1, wsem0, wsem1):
        wid = lax.axis_index("s") * NUM_CORES + lax.axis_index("c")
        base = wid * nb
        gsem = (gsem0, gsem1)
        wsem = (wsem0, wsem1)
        pltpu.sync_copy(idx_hbm.at[pl.ds(base * SEQ_PAD, nb * SEQ_PAD)], idx_v)

        def g_main(j, b):
            return pltpu.make_async_copy(
                table_hbm.at[idx_v.at[pl.ds(j * SEQ_PAD, 64)]],
                rows_v.at[b], gsem[b])

        def g_tail(j, b):
            return pltpu.make_async_copy(
                table_hbm.at[idx_v.at[pl.ds(j * SEQ_PAD + 64, 1)]],
                tail_v.at[b], gsem[b])

        def g_start(j, b):
            g_main(j, b).start()
            g_tail(j, b).start()

        def g_wait(j, b):
            g_main(j, b).wait()
            g_tail(j, b).wait()

        def w_main(j, b):
            return pltpu.make_async_copy(
                rows_v.at[b], out_hbm.at[base + j, pl.ds(0, 64)], wsem[b])

        def w_tail(j, b):
            return pltpu.make_async_copy(
                tail_v.at[b], out_hbm.at[base + j, pl.ds(64, 1)], wsem[b])

        def w_start(j, b):
            w_main(j, b).start()
            w_tail(j, b).start()

        def w_wait(j, b):
            w_main(j, b).wait()
            w_tail(j, b).wait()

        g_start(0, 0)
        g_start(1, 1)

        def body(i, carry):
            j0 = 2 * i
            more = j0 + 2 < nb
            g_wait(j0, 0)
            w_start(j0, 0)
            g_wait(j0 + 1, 1)
            w_start(j0 + 1, 1)
            w_wait(j0, 0)
            @pl.when(more)
            def _():
                g_start(j0 + 2, 0)
            w_wait(j0 + 1, 1)
            @pl.when(more)
            def _():
                g_start(j0 + 3, 1)
            return carry

        lax.fori_loop(0, nb // 2, body, 0)

    return gather


def kernel(board_tokens, color_tokens, piece_table, color_table, square_table,
           gamma, beta):
    bsz = board_tokens.shape[0]
    table3, idx = _build_table_and_idx(
        piece_table, color_table, square_table, gamma, beta,
        board_tokens.astype(jnp.int32), color_tokens.astype(jnp.int32))
    table2 = table3.reshape(N_ROWS, D_MODEL)
    return _make_gather(bsz)(table2, idx.reshape(bsz * SEQ_PAD))
